# Initial kernel scaffold; baseline (speedup 1.0000x reference)
import jax
import jax.numpy as jnp
from jax.experimental import pallas as pl


def kernel(x, edge_index, graph_ids, self_feat, g1_w1, g1_b1, g1_bn_g, g1_bn_b, g1_w2, g1_b2, bn1_g, bn1_b, g2_w1, g2_b1, g2_bn_g, g2_bn_b, g2_w2, g2_b2, bn2_g, bn2_b, fc1_w, fc1_b, fc2_w, fc2_b):
    raise NotImplementedError("write your pallas kernel here")


# speedup vs baseline: 2.2713x; 2.2713x over previous
import functools

import jax
import jax.numpy as jnp
from jax import lax
from jax.experimental import pallas as pl
from jax.experimental.pallas import tpu as pltpu
from jax.experimental.pallas import tpu_sc as plsc

N = 10000
E = 320000
DIN = 128
B = 64
DSF = 16
DOUT = 10

NC = 2
NS = 16
NW = NC * NS
K = 128
CHUNKS = -(-E // (NW * K))
EPW = CHUNKS * K
EPAD = EPW * NW
RPS = 640
NPAD = RPS * NS
D = 128


def _sc_partial_agg(feat, srcp, dstp):
    mesh = plsc.VectorSubcoreMesh(
        core_axis_name="c", subcore_axis_name="s", num_cores=NC, num_subcores=NS
    )

    @functools.partial(
        pl.kernel,
        out_type=jax.ShapeDtypeStruct((NC * NPAD, D), jnp.float32),
        mesh=mesh,
        scratch_types=[
            pltpu.VMEM((K,), jnp.int32),
            pltpu.VMEM((K,), jnp.int32),
            pltpu.VMEM((K, D), jnp.float32),
            pltpu.VMEM((8, D), jnp.float32),
            pltpu.VMEM_SHARED((NPAD, D), jnp.float32),
            pltpu.SemaphoreType.DMA,
        ],
    )
    def agg_kernel(feat_hbm, src_hbm, dst_hbm, out_hbm,
                   src_v, dst_v, rows_v, zer_v, acc_sh, sem):
        cid = lax.axis_index("c")
        sid = lax.axis_index("s")
        wid = sid * NC + cid

        @pl.loop(0, 8)
        def _(r):
            @pl.loop(0, D, step=16)
            def _(c):
                zer_v[r, pl.ds(c, 16)] = jnp.zeros((16,), jnp.float32)

        base = sid * RPS
        pltpu.sync_copy(zer_v, acc_sh.at[pl.ds(base, 8)])
        have = 8
        while have < RPS:
            step = min(have, RPS - have)
            pltpu.sync_copy(acc_sh.at[pl.ds(base, step)],
                            acc_sh.at[pl.ds(base + have, step)])
            have += step
        plsc.subcore_barrier()

        @pl.loop(0, CHUNKS)
        def _(j):
            base = wid * EPW + j * K
            pltpu.sync_copy(src_hbm.at[pl.ds(base, K)], src_v)
            pltpu.sync_copy(dst_hbm.at[pl.ds(base, K)], dst_v)
            pltpu.async_copy(feat_hbm.at[src_v], rows_v, sem).wait()
            pltpu.sync_copy(rows_v, acc_sh.at[dst_v], add=True)

        plsc.subcore_barrier()
        pltpu.sync_copy(
            acc_sh.at[pl.ds(sid * RPS, RPS)],
            out_hbm.at[pl.ds(cid * NPAD + sid * RPS, RPS)],
        )

    return agg_kernel(feat, srcp, dstp)


def _bn_relu(h, gamma, beta):
    m = jnp.mean(h, axis=0)
    v = jnp.mean((h - m) ** 2, axis=0)
    return jnp.maximum((h - m) * lax.rsqrt(v + 1e-5) * gamma + beta, 0.0)


def _tc_layer1(x, part, w1, b1, bng, bnb, w2, b2, g1, bb1):
    def body(x_ref, p_ref, w1_ref, b1_ref, bng_ref, bnb_ref,
             w2_ref, b2_ref, g1_ref, bb1_ref, out_ref):
        a = x_ref[...] + p_ref[0:N, :] + p_ref[NPAD:NPAD + N, :]
        h = jnp.dot(a, w1_ref[...], preferred_element_type=jnp.float32) + b1_ref[...]
        h = _bn_relu(h, bng_ref[...], bnb_ref[...])
        h = jnp.dot(h, w2_ref[...], preferred_element_type=jnp.float32) + b2_ref[...]
        h = _bn_relu(h, g1_ref[...], bb1_ref[...])
        out_ref[...] = jnp.concatenate(
            [h, jnp.zeros((N, D - h.shape[1]), jnp.float32)], axis=1
        )

    return pl.pallas_call(
        body, out_shape=jax.ShapeDtypeStruct((N, D), jnp.float32)
    )(x, part, w1, b1, bng, bnb, w2, b2, g1, bb1)


def _tc_layer2(h1, part, gids, sf, w1, b1, bng, bnb, w2, b2, g2, bb2,
               f1w, f1b, f2w, f2b):
    def body(h_ref, p_ref, gid_ref, sf_ref, w1_ref, b1_ref, bng_ref, bnb_ref,
             w2_ref, b2_ref, g2_ref, bb2_ref, f1w_ref, f1b_ref, f2w_ref,
             f2b_ref, out_ref):
        a = h_ref[...] + p_ref[0:N, :] + p_ref[NPAD:NPAD + N, :]
        a = a[:, 0:100]
        h = jnp.dot(a, w1_ref[...], preferred_element_type=jnp.float32) + b1_ref[...]
        h = _bn_relu(h, bng_ref[...], bnb_ref[...])
        h = jnp.dot(h, w2_ref[...], preferred_element_type=jnp.float32) + b2_ref[...]
        h = _bn_relu(h, g2_ref[...], bb2_ref[...])
        onehot = (gid_ref[...] == lax.broadcasted_iota(jnp.int32, (1, B), 1))
        onehot = onehot.astype(jnp.float32)
        sums = lax.dot_general(
            onehot, h, (((0,), (0,)), ((), ())),
            preferred_element_type=jnp.float32,
        )
        counts = jnp.sum(onehot, axis=0)
        hg = sums / jnp.maximum(counts, 1.0)[:, None]
        hg = jnp.concatenate([hg, sf_ref[...]], axis=1)
        o = jnp.maximum(
            jnp.dot(hg, f1w_ref[...], preferred_element_type=jnp.float32)
            + f1b_ref[...], 0.0)
        out_ref[...] = (
            jnp.dot(o, f2w_ref[...], preferred_element_type=jnp.float32)
            + f2b_ref[...]
        )

    return pl.pallas_call(
        body, out_shape=jax.ShapeDtypeStruct((B, DOUT), jnp.float32)
    )(h1, part, gids, sf, w1, b1, bng, bnb, w2, b2, g2, bb2, f1w, f1b, f2w, f2b)


def kernel(x, edge_index, graph_ids, self_feat,
           g1_w1, g1_b1, g1_bn_g, g1_bn_b, g1_w2, g1_b2, bn1_g, bn1_b,
           g2_w1, g2_b1, g2_bn_g, g2_bn_b, g2_w2, g2_b2, bn2_g, bn2_b,
           fc1_w, fc1_b, fc2_w, fc2_b):
    pad = EPAD - E
    srcp = jnp.concatenate([edge_index[0], jnp.zeros((pad,), jnp.int32)])
    dstp = jnp.concatenate([edge_index[1], jnp.full((pad,), N, jnp.int32)])

    part1 = _sc_partial_agg(x, srcp, dstp)
    h1 = _tc_layer1(x, part1, g1_w1, g1_b1, g1_bn_g, g1_bn_b,
                    g1_w2, g1_b2, bn1_g, bn1_b)
    part2 = _sc_partial_agg(h1, srcp, dstp)
    out = _tc_layer2(h1, part2, graph_ids.reshape(N, 1), self_feat,
                     g2_w1, g2_b1, g2_bn_g, g2_bn_b, g2_w2, g2_b2,
                     bn2_g, bn2_b, fc1_w, fc1_b, fc2_w, fc2_b)
    return out
